# baseline (speedup 1.0000x reference)
import functools

import jax
import jax.numpy as jnp
from jax import lax
from jax.experimental import pallas as pl
from jax.experimental.pallas import tpu as pltpu
from jax.experimental.pallas import tpu_sc as plsc

_D = _H = _W = 24
_C = 128
_L = 16
_CC = _C // _L
_NW = 32
_ROW = _W * _C
_SLAB = _H * _ROW
_JROWS = 4
_JWORDS = _JROWS * _ROW
_NRING = 8


_DBLK = 4


def _tp_body(x_ref, o_ref):
    for d in range(_DBLK):
        o_ref[d] = x_ref[0, :, d].reshape(_C, _H * _W).T


def _tc_channel_minor(fm):
    B, C, D, H, W = fm.shape
    out = pl.pallas_call(
        _tp_body,
        grid=(B, D // _DBLK),
        in_specs=[pl.BlockSpec((1, C, _DBLK, H, W),
                               lambda b, j: (b, 0, j, 0, 0))],
        out_specs=pl.BlockSpec((_DBLK, H * W, C),
                               lambda b, j: (b * (D // _DBLK) + j, 0, 0)),
        out_shape=jax.ShapeDtypeStruct((B * D, H * W, C), jnp.float32),
    )(fm)
    return out.reshape(B * D * H * W * C)


def _ot_body(x_ref, o_ref):
    o_ref[...] = jnp.swapaxes(x_ref[...], 2, 3)


def _tc_oct_minor(flat, B, N):
    x = flat.reshape(B * N, _CC, 8, _L)
    out = pl.pallas_call(
        _ot_body,
        grid=(1,),
        in_specs=[pl.BlockSpec((B * N, _CC, 8, _L), lambda i: (0, 0, 0, 0))],
        out_specs=pl.BlockSpec((B * N, _CC, _L, 8), lambda i: (0, 0, 0, 0)),
        out_shape=jax.ShapeDtypeStruct((B * N, _CC, _L, 8), jnp.float32),
    )(x)
    return out


def _build_sc_kernel(num_props):
    mesh = plsc.VectorSubcoreMesh(core_axis_name="c", subcore_axis_name="s")
    units_per_w = num_props // _NW

    @functools.partial(
        pl.kernel,
        mesh=mesh,
        out_type=jax.ShapeDtypeStruct((num_props * 8 * _C,), jnp.float32),
        scratch_types=[
            pltpu.VMEM((num_props * _L,), jnp.int32),
            pltpu.VMEM((_NRING * _JWORDS + 512,), jnp.float32),
            pltpu.VMEM((8 * _C,), jnp.float32),
            pltpu.VMEM((units_per_w * 8 * _C,), jnp.float32),
        ] + [pltpu.SemaphoreType.DMA] * (_NRING + 1),
    )
    def k(fm_hbm, par_hbm, out_hbm, par_v, ring, acc, outbuf, *sems_all):
        sems = sems_all[:_NRING]
        osem = sems_all[_NRING]
        wid = lax.axis_index("s") * 2 + lax.axis_index("c")
        pltpu.sync_copy(par_hbm, par_v)
        neg = jnp.full((_L,), -jnp.inf, jnp.float32)
        i32 = jnp.int32

        def load_pv(t):
            ts = jnp.minimum(t, units_per_w - 1)
            pn = ts * _NW + wid
            b = lax.shift_right_logical(pn, 6)
            return par_v[pl.ds(pn * _L, _L)], b

        def nk_of(pv):
            nh = pv[6] + pv[7] - pv[4]
            return lax.shift_right_logical(nh + _JROWS - 1, 2)

        def h0_of(pv, kk):
            return jnp.minimum(pv[4] + kk * _JROWS, _H - _JROWS)

        def issue_job_static(b, d, h0, s):
            src = (b * _D + d) * _SLAB + h0 * _ROW
            pltpu.async_copy(
                fm_hbm.at[pl.ds(src, _JWORDS)],
                ring.at[pl.ds(s * _JWORDS, _JWORDS)],
                sems[s])

        def wait_job_static(s):
            pltpu.make_async_copy(
                fm_hbm.at[pl.ds(0, _JWORDS)],
                ring.at[pl.ds(s * _JWORDS, _JWORDS)],
                sems[s]).wait()

        def adv(t, d, kk, pv, b):
            k2 = kk + 1
            roll_k = k2 >= nk_of(pv)
            k3 = jnp.where(roll_k, 0, k2)
            d2 = jnp.where(roll_k, d + 1, d)
            d_hi = pv[2] + pv[3]
            roll_u = jnp.logical_and(roll_k, d2 >= d_hi)
            t2 = jnp.where(roll_u, t + 1, t)
            pv2, b2 = load_pv(t2)
            pv3 = jnp.where(roll_u, pv2, pv)
            b3 = jnp.where(roll_u, b2, b)
            d3 = jnp.where(roll_u, pv3[0], d2)
            return t2, d3, k3, pv3, b3, roll_u

        def wmax8(hb, sw, lw):
            w_end = sw + lw
            nch = lax.shift_right_logical(lw + 3, 2)

            def chunk(j, ms):
                w0 = sw + j * 4
                base = hb + w0 * _C
                out = []
                for cc in range(_CC):
                    m = ms[cc]
                    for e in range(4):
                        x = ring[pl.ds(base + e * _C + cc * _L, _L)]
                        if e == 0:
                            m = jnp.maximum(m, x)
                        else:
                            m = jnp.maximum(
                                m, jnp.where(w0 + e < w_end, x, neg))
                    out.append(m)
                return tuple(out)

            ms = lax.fori_loop(0, nch, chunk, (neg,) * _CC)
            return ms

        def compute_job(pv, d, kk, slot_idx):
            h0 = h0_of(pv, kk)
            base_p = slot_idx * _JWORDS - h0 * _ROW
            sd0, ld0, sd1 = pv[0], pv[1], pv[2]
            in_d = (d < sd0 + ld0, d >= sd1)
            for bd in range(2):
                @pl.when(in_d[bd])
                def _(bd=bd):
                    for bh in range(2):
                        sh = pv[4] if bh == 0 else pv[6]
                        lh = pv[5] if bh == 0 else pv[7]
                        hlo = jnp.maximum(sh, h0)
                        hhi = jnp.minimum(sh + lh, h0 + _JROWS)

                        def h_body(h, c2, bd=bd, bh=bh):
                            hb = base_p + h * _ROW
                            for bw in range(2):
                                sw = pv[8] if bw == 0 else pv[10]
                                lw = pv[9] if bw == 0 else pv[11]
                                ms = wmax8(hb, sw, lw)
                                o = bd * 4 + bh * 2 + bw
                                for cc in range(_CC):
                                    slot = (cc * 8 + o) * _L
                                    acc[pl.ds(slot, _L)] = jnp.maximum(
                                        acc[pl.ds(slot, _L)], ms[cc])
                            return c2

                        lax.fori_loop(hlo, hhi, h_body, 0)
            return None

        def flush_unit(t):
            pn = t * _NW + wid
            for o in range(8 * _CC):
                outbuf[pl.ds(t * 1024 + o * _L, _L)] = acc[pl.ds(o * _L, _L)]
            pltpu.async_copy(
                outbuf.at[pl.ds(t * 1024, 1024)],
                out_hbm.at[pl.ds(pn * 1024, 1024)],
                osem)
            for o in range(8 * _CC):
                acc[pl.ds(o * _L, _L)] = neg

        for o in range(8 * _CC):
            acc[pl.ds(o * _L, _L)] = neg

        pv0, b0 = load_pv(0)
        ti, di, ki, pvi, bi = i32(0), pv0[0], i32(0), pv0, b0
        for s in range(_NRING):
            issue_job_static(bi, di, h0_of(pvi, ki), s)
            ti, di, ki, pvi, bi, _ = adv(ti, di, ki, pvi, bi)

        def count_body(t, tot):
            pv, _ = load_pv(t)
            nd = pv[2] + pv[3] - pv[0]
            return tot + nd * nk_of(pv)

        total_jobs = lax.fori_loop(0, units_per_w, count_body, i32(0))

        def body(cs, st):
            tc, dc, kc, pvc, bc, ti, di, ki, pvi, bi = st
            slot = lax.bitwise_and(cs, i32(_NRING - 1))
            more = ti < units_per_w
            h0i = h0_of(pvi, ki)
            for s in range(_NRING):
                @pl.when(slot == s)
                def _(s=s):
                    wait_job_static(s)
            compute_job(pvc, dc, kc, slot)
            for s in range(_NRING):
                @pl.when(jnp.logical_and(slot == s, more))
                def _(s=s):
                    issue_job_static(bi, di, h0i, s)
            ti2, di2, ki2, pvi2, bi2, _ = adv(ti, di, ki, pvi, bi)
            tc2, dc2, kc2, pvc2, bc2, rolled = adv(tc, dc, kc, pvc, bc)

            @pl.when(rolled)
            def _():
                flush_unit(tc)

            return (tc2, dc2, kc2, pvc2, bc2,
                    ti2, di2, ki2, pvi2, bi2)

        pvc0, bc0 = load_pv(0)
        lax.fori_loop(0, total_jobs, body, (i32(0), pvc0[0], i32(0), pvc0,
                                            bc0, ti, di, ki, pvi, bi))

        def drain(t, carry):
            pltpu.make_async_copy(
                outbuf.at[pl.ds(0, 1024)],
                out_hbm.at[pl.ds(0, 1024)],
                osem).wait()
            return carry

        lax.fori_loop(0, units_per_w, drain, 0)

    return k


def kernel(fm, corners, scale):
    B, C, D, H, W = fm.shape
    N = corners.shape[1]

    c32 = corners.astype(jnp.int32)
    p1 = jnp.clip(c32[:, :, 0, :] // scale, 0, 21)
    p2r = c32[:, :, 1, :] // scale
    p2 = jnp.where(p2r - p1 >= 2, p2r, p1 + 2)
    n = p2 - p1
    s0, s1 = p1, p1 + n // 2
    l0, l1 = (n + 1) // 2, n - n // 2
    pr = jnp.stack(
        [s0[..., 0], l0[..., 0], s1[..., 0], l1[..., 0],
         s0[..., 1], l0[..., 1], s1[..., 1], l1[..., 1],
         s0[..., 2], l0[..., 2], s1[..., 2], l1[..., 2]], axis=-1)
    params = jnp.concatenate(
        [pr, jnp.zeros((B, N, 4), jnp.int32)], axis=-1).reshape(B * N * _L)

    fm2 = _tc_channel_minor(fm)
    out = _build_sc_kernel(B * N)(fm2, params)
    out = _tc_oct_minor(out, B, N).reshape(B, N, C, 2, 2, 2)
    return out


# speedup vs baseline: 1.4862x; 1.4862x over previous
import functools

import jax
import jax.numpy as jnp
from jax import lax
from jax.experimental import pallas as pl
from jax.experimental.pallas import tpu as pltpu
from jax.experimental.pallas import tpu_sc as plsc

_D = _H = _W = 24
_C = 128
_L = 16
_CC = _C // _L
_NW = 32
_ROW = _W * _C
_SLAB = _H * _ROW
_JROWS = 4
_JWORDS = _JROWS * _ROW
_NRING = 8


def _tp_body(x_ref, o_ref):
    o_ref[0] = x_ref[0].T


def _tc_channel_minor(fm):
    B, C, D, H, W = fm.shape
    v = D * H * W
    out = pl.pallas_call(
        _tp_body,
        grid=(B,),
        in_specs=[pl.BlockSpec((1, C, v), lambda b: (b, 0, 0))],
        out_specs=pl.BlockSpec((1, v, C), lambda b: (b, 0, 0)),
        out_shape=jax.ShapeDtypeStruct((B, v, C), jnp.float32),
    )(fm.reshape(B, C, v))
    return out.reshape(B * v * C)


def _ot_body(x_ref, o_ref):
    o_ref[...] = jnp.swapaxes(x_ref[...], 2, 3)


def _tc_oct_minor(flat, B, N):
    x = flat.reshape(B * N, _CC, 8, _L)
    out = pl.pallas_call(
        _ot_body,
        grid=(1,),
        in_specs=[pl.BlockSpec((B * N, _CC, 8, _L), lambda i: (0, 0, 0, 0))],
        out_specs=pl.BlockSpec((B * N, _CC, _L, 8), lambda i: (0, 0, 0, 0)),
        out_shape=jax.ShapeDtypeStruct((B * N, _CC, _L, 8), jnp.float32),
    )(x)
    return out


def _build_sc_kernel(num_props):
    mesh = plsc.VectorSubcoreMesh(core_axis_name="c", subcore_axis_name="s")
    units_per_w = num_props // _NW

    @functools.partial(
        pl.kernel,
        mesh=mesh,
        out_type=jax.ShapeDtypeStruct((num_props * 8 * _C,), jnp.float32),
        scratch_types=[
            pltpu.VMEM((num_props * _L,), jnp.int32),
            pltpu.VMEM((_NRING * _JWORDS + 512,), jnp.float32),
            pltpu.VMEM((8 * _C,), jnp.float32),
            pltpu.VMEM((units_per_w * 8 * _C,), jnp.float32),
        ] + [pltpu.SemaphoreType.DMA] * (_NRING + 1),
    )
    def k(fm_hbm, par_hbm, out_hbm, par_v, ring, acc, outbuf, *sems_all):
        sems = sems_all[:_NRING]
        osem = sems_all[_NRING]
        wid = lax.axis_index("s") * 2 + lax.axis_index("c")
        pltpu.sync_copy(par_hbm, par_v)
        neg = jnp.full((_L,), -jnp.inf, jnp.float32)
        i32 = jnp.int32

        def load_pv(t):
            ts = jnp.minimum(t, units_per_w - 1)
            pn = ts * _NW + wid
            b = lax.shift_right_logical(pn, 6)
            return par_v[pl.ds(pn * _L, _L)], b

        def nk_of(pv):
            nh = pv[6] + pv[7] - pv[4]
            return lax.shift_right_logical(nh + _JROWS - 1, 2)

        def h0_of(pv, kk):
            return jnp.minimum(pv[4] + kk * _JROWS, _H - _JROWS)

        def issue_job_static(b, d, h0, s):
            src = (b * _D + d) * _SLAB + h0 * _ROW
            pltpu.async_copy(
                fm_hbm.at[pl.ds(src, _JWORDS)],
                ring.at[pl.ds(s * _JWORDS, _JWORDS)],
                sems[s])

        def wait_job_static(s):
            pltpu.make_async_copy(
                fm_hbm.at[pl.ds(0, _JWORDS)],
                ring.at[pl.ds(s * _JWORDS, _JWORDS)],
                sems[s]).wait()

        def adv(t, d, kk, pv, b):
            k2 = kk + 1
            roll_k = k2 >= nk_of(pv)
            k3 = jnp.where(roll_k, 0, k2)
            d2 = jnp.where(roll_k, d + 1, d)
            d_hi = pv[2] + pv[3]
            roll_u = jnp.logical_and(roll_k, d2 >= d_hi)
            t2 = jnp.where(roll_u, t + 1, t)
            pv2, b2 = load_pv(t2)
            pv3 = jnp.where(roll_u, pv2, pv)
            b3 = jnp.where(roll_u, b2, b)
            d3 = jnp.where(roll_u, pv3[0], d2)
            return t2, d3, k3, pv3, b3, roll_u

        def wmax8(hb, sw, lw):
            w_end = sw + lw
            nch = lax.shift_right_logical(lw + 3, 2)

            def chunk(j, ms):
                w0 = sw + j * 4
                base = hb + w0 * _C
                out = []
                for cc in range(_CC):
                    m = ms[cc]
                    for e in range(4):
                        x = ring[pl.ds(base + e * _C + cc * _L, _L)]
                        if e == 0:
                            m = jnp.maximum(m, x)
                        else:
                            m = jnp.maximum(
                                m, jnp.where(w0 + e < w_end, x, neg))
                    out.append(m)
                return tuple(out)

            ms = lax.fori_loop(0, nch, chunk, (neg,) * _CC)
            return ms

        def compute_job(pv, d, kk, slot_idx):
            h0 = h0_of(pv, kk)
            base_p = slot_idx * _JWORDS - h0 * _ROW
            sd0, ld0, sd1 = pv[0], pv[1], pv[2]
            in_d = (d < sd0 + ld0, d >= sd1)
            for bd in range(2):
                @pl.when(in_d[bd])
                def _(bd=bd):
                    for bh in range(2):
                        sh = pv[4] if bh == 0 else pv[6]
                        lh = pv[5] if bh == 0 else pv[7]
                        hlo = jnp.maximum(sh, h0)
                        hhi = jnp.minimum(sh + lh, h0 + _JROWS)

                        def h_body(h, c2, bd=bd, bh=bh):
                            hb = base_p + h * _ROW
                            for bw in range(2):
                                sw = pv[8] if bw == 0 else pv[10]
                                lw = pv[9] if bw == 0 else pv[11]
                                ms = wmax8(hb, sw, lw)
                                o = bd * 4 + bh * 2 + bw
                                for cc in range(_CC):
                                    slot = (cc * 8 + o) * _L
                                    acc[pl.ds(slot, _L)] = jnp.maximum(
                                        acc[pl.ds(slot, _L)], ms[cc])
                            return c2

                        lax.fori_loop(hlo, hhi, h_body, 0)
            return None

        def flush_unit(t):
            pn = t * _NW + wid
            for o in range(8 * _CC):
                outbuf[pl.ds(t * 1024 + o * _L, _L)] = acc[pl.ds(o * _L, _L)]
            pltpu.async_copy(
                outbuf.at[pl.ds(t * 1024, 1024)],
                out_hbm.at[pl.ds(pn * 1024, 1024)],
                osem)
            for o in range(8 * _CC):
                acc[pl.ds(o * _L, _L)] = neg

        for o in range(8 * _CC):
            acc[pl.ds(o * _L, _L)] = neg

        pv0, b0 = load_pv(0)
        ti, di, ki, pvi, bi = i32(0), pv0[0], i32(0), pv0, b0
        for s in range(_NRING):
            issue_job_static(bi, di, h0_of(pvi, ki), s)
            ti, di, ki, pvi, bi, _ = adv(ti, di, ki, pvi, bi)

        def count_body(t, tot):
            pv, _ = load_pv(t)
            nd = pv[2] + pv[3] - pv[0]
            return tot + nd * nk_of(pv)

        total_jobs = lax.fori_loop(0, units_per_w, count_body, i32(0))

        def body(cs, st):
            tc, dc, kc, pvc, bc, ti, di, ki, pvi, bi = st
            slot = lax.bitwise_and(cs, i32(_NRING - 1))
            more = ti < units_per_w
            h0i = h0_of(pvi, ki)
            for s in range(_NRING):
                @pl.when(slot == s)
                def _(s=s):
                    wait_job_static(s)
            compute_job(pvc, dc, kc, slot)
            for s in range(_NRING):
                @pl.when(jnp.logical_and(slot == s, more))
                def _(s=s):
                    issue_job_static(bi, di, h0i, s)
            ti2, di2, ki2, pvi2, bi2, _ = adv(ti, di, ki, pvi, bi)
            tc2, dc2, kc2, pvc2, bc2, rolled = adv(tc, dc, kc, pvc, bc)

            @pl.when(rolled)
            def _():
                flush_unit(tc)

            return (tc2, dc2, kc2, pvc2, bc2,
                    ti2, di2, ki2, pvi2, bi2)

        pvc0, bc0 = load_pv(0)
        lax.fori_loop(0, total_jobs, body, (i32(0), pvc0[0], i32(0), pvc0,
                                            bc0, ti, di, ki, pvi, bi))

        def drain(t, carry):
            pltpu.make_async_copy(
                outbuf.at[pl.ds(0, 1024)],
                out_hbm.at[pl.ds(0, 1024)],
                osem).wait()
            return carry

        lax.fori_loop(0, units_per_w, drain, 0)

    return k


def kernel(fm, corners, scale):
    B, C, D, H, W = fm.shape
    N = corners.shape[1]

    c32 = corners.astype(jnp.int32)
    p1 = jnp.clip(c32[:, :, 0, :] // scale, 0, 21)
    p2r = c32[:, :, 1, :] // scale
    p2 = jnp.where(p2r - p1 >= 2, p2r, p1 + 2)
    n = p2 - p1
    s0, s1 = p1, p1 + n // 2
    l0, l1 = (n + 1) // 2, n - n // 2
    pr = jnp.stack(
        [s0[..., 0], l0[..., 0], s1[..., 0], l1[..., 0],
         s0[..., 1], l0[..., 1], s1[..., 1], l1[..., 1],
         s0[..., 2], l0[..., 2], s1[..., 2], l1[..., 2]], axis=-1)
    params = jnp.concatenate(
        [pr, jnp.zeros((B, N, 4), jnp.int32)], axis=-1).reshape(B * N * _L)

    fm2 = _tc_channel_minor(fm)
    out = _build_sc_kernel(B * N)(fm2, params)
    out = _tc_oct_minor(out, B, N).reshape(B, N, C, 2, 2, 2)
    return out
